# baseline (speedup 1.0000x reference)
import jax
import jax.numpy as jnp
from jax.experimental import pallas as pl
from jax.experimental.pallas import tpu as pltpu

_TOK_ROWS = 128
_TOK_COLS = 128
_D = 2048
_R = 4


def _body(mask_ref, out_ref, a_ref):
    i = pl.program_id(0)

    @pl.when(i == 0)
    def _init():
        m = mask_ref[...].astype(jnp.float32)
        r_iota = jax.lax.broadcasted_iota(jnp.int32, (_TOK_ROWS, _TOK_COLS), 0)
        c_iota = jax.lax.broadcasted_iota(jnp.int32, (_TOK_ROWS, _TOK_COLS), 1)
        upper = (r_iota <= c_iota).astype(jnp.float32)
        lower = (r_iota >= c_iota).astype(jnp.float32)
        row_cs = jnp.dot(m, upper, preferred_element_type=jnp.float32)
        row_tot = row_cs[:, _TOK_COLS - 1:_TOK_COLS]
        row_off = jnp.dot(lower, row_tot,
                          preferred_element_type=jnp.float32) - row_tot
        a_ref[...] = (row_cs + row_off - m).astype(jnp.int32) * _D - 1

    a_blk = a_ref[pl.ds(i * _R, _R), :]
    m_blk = mask_ref[pl.ds(i * _R, _R), :]
    shp = (_R, _TOK_COLS, _D // 128, 128)
    j1 = (jax.lax.broadcasted_iota(jnp.int32, shp, 2) * 128
          + jax.lax.broadcasted_iota(jnp.int32, shp, 3) + 1)
    out_ref[...] = (a_blk[:, :, None, None]
                    + m_blk[:, :, None, None] * j1).astype(jnp.float32)


def kernel(inputs_embeds, images_seq_mask):
    del inputs_embeds
    m2 = images_seq_mask.reshape(_TOK_ROWS, _TOK_COLS).astype(jnp.int32)
    out = pl.pallas_call(
        _body,
        grid=(_TOK_ROWS // _R,),
        in_specs=[
            pl.BlockSpec((_TOK_ROWS, _TOK_COLS), lambda i: (0, 0)),
        ],
        out_specs=pl.BlockSpec((_R, _TOK_COLS, _D // 128, 128),
                               lambda i: (i, 0, 0, 0)),
        out_shape=jax.ShapeDtypeStruct(
            (_TOK_ROWS, _TOK_COLS, _D // 128, 128), jnp.float32),
        scratch_shapes=[pltpu.VMEM((_TOK_ROWS, _TOK_COLS), jnp.int32)],
    )(m2)
    return out.reshape(-1)


# speedup vs baseline: 1.0681x; 1.0681x over previous
import jax
import jax.numpy as jnp
from jax.experimental import pallas as pl
from jax.experimental.pallas import tpu as pltpu

_TOK_ROWS = 128
_TOK_COLS = 128
_D = 2048
_R = 8


def _body(mask_ref, out_ref):
    i = pl.program_id(0)

    m = mask_ref[...].astype(jnp.float32)
    m_blk = mask_ref[pl.ds(i * _R, _R), :]
    m_blk_f = m_blk.astype(jnp.float32)
    r_iota = jax.lax.broadcasted_iota(jnp.int32, (_R, _TOK_COLS), 0) + i * _R
    c_iota = jax.lax.broadcasted_iota(jnp.int32, (_R, _TOK_COLS), 1)
    upper = (jax.lax.broadcasted_iota(jnp.int32, (_TOK_COLS, _TOK_COLS), 0)
             <= jax.lax.broadcasted_iota(
                 jnp.int32, (_TOK_COLS, _TOK_COLS), 1)).astype(jnp.float32)
    lower_blk = (r_iota >= c_iota).astype(jnp.float32)
    row_cs_blk = jnp.dot(m_blk_f, upper, preferred_element_type=jnp.float32)
    row_tot_blk = row_cs_blk[:, _TOK_COLS - 1:_TOK_COLS]
    ones_col = jnp.ones((_TOK_COLS, 1), jnp.float32)
    row_tot = jnp.dot(m, ones_col, preferred_element_type=jnp.float32)
    row_off_blk = jnp.dot(lower_blk, row_tot,
                          preferred_element_type=jnp.float32) - row_tot_blk
    a_blk = (row_cs_blk + row_off_blk - m_blk_f).astype(jnp.int32) * _D - 1
    shp = (_R, _TOK_COLS, _D // 128, 128)
    j1 = (jax.lax.broadcasted_iota(jnp.int32, shp, 2) * 128
          + jax.lax.broadcasted_iota(jnp.int32, shp, 3) + 1)
    out_ref[...] = (a_blk[:, :, None, None]
                    + m_blk[:, :, None, None] * j1).astype(jnp.float32)


def kernel(inputs_embeds, images_seq_mask):
    del inputs_embeds
    m2 = images_seq_mask.reshape(_TOK_ROWS, _TOK_COLS).astype(jnp.int32)
    out = pl.pallas_call(
        _body,
        grid=(_TOK_ROWS // _R,),
        in_specs=[
            pl.BlockSpec((_TOK_ROWS, _TOK_COLS), lambda i: (0, 0)),
        ],
        out_specs=pl.BlockSpec((_R, _TOK_COLS, _D // 128, 128),
                               lambda i: (i, 0, 0, 0)),
        out_shape=jax.ShapeDtypeStruct(
            (_TOK_ROWS, _TOK_COLS, _D // 128, 128), jnp.float32),
        compiler_params=pltpu.CompilerParams(
            dimension_semantics=("parallel",)),
    )(m2)
    return out.reshape(-1)
